# baseline (speedup 1.0000x reference)
import jax
import jax.numpy as jnp
from jax.experimental import pallas as pl

_BLK = 1024


def _encode_block(p_ref, wk_ref, bk_ref, wv_ref, bv_ref, k_ref, v_ref):
    p = p_ref[...]
    z = jnp.zeros((64, 64), jnp.float32)
    wkt = wk_ref[...].T
    wvt = wv_ref[...].T
    wkd = jnp.concatenate(
        [jnp.concatenate([wkt, z], 1), jnp.concatenate([z, wkt], 1)], 0
    )
    wvd = jnp.concatenate(
        [jnp.concatenate([wvt, z], 1), jnp.concatenate([z, wvt], 1)], 0
    )
    bkd = jnp.concatenate([bk_ref[...], bk_ref[...]], 1)
    bvd = jnp.concatenate([bv_ref[...], bv_ref[...]], 1)
    k_ref[...] = jnp.dot(p, wkd, preferred_element_type=jnp.float32) + bkd
    v_ref[...] = jnp.dot(p, wvd, preferred_element_type=jnp.float32) + bvd


def kernel(x, labels, prototype_vectors, Wk, bk, Wv, bv):
    n, d = prototype_vectors.shape
    a = Wk.shape[0]
    p2 = prototype_vectors.reshape(n // 2, 2 * d)
    bk2 = bk.reshape(1, a)
    bv2 = bv.reshape(1, a)
    k2, v2 = pl.pallas_call(
        _encode_block,
        grid=((n // 2) // _BLK,),
        in_specs=[
            pl.BlockSpec((_BLK, 2 * d), lambda i: (i, 0)),
            pl.BlockSpec((a, d), lambda i: (0, 0)),
            pl.BlockSpec((1, a), lambda i: (0, 0)),
            pl.BlockSpec((a, d), lambda i: (0, 0)),
            pl.BlockSpec((1, a), lambda i: (0, 0)),
        ],
        out_specs=[
            pl.BlockSpec((_BLK, 2 * a), lambda i: (i, 0)),
            pl.BlockSpec((_BLK, 2 * a), lambda i: (i, 0)),
        ],
        out_shape=[
            jax.ShapeDtypeStruct((n // 2, 2 * a), jnp.float32),
            jax.ShapeDtypeStruct((n // 2, 2 * a), jnp.float32),
        ],
    )(p2, Wk, bk2, Wv, bv2)
    return (k2.reshape(n, a), v2.reshape(n, a))


# speedup vs baseline: 1.0130x; 1.0130x over previous
import jax
import jax.numpy as jnp
from jax.experimental import pallas as pl
from jax.experimental.pallas import tpu as pltpu

_N2 = 4096
_C = 8
_R = _N2 // _C


def _encode(p_hbm, wk_ref, bk_ref, wv_ref, bv_ref, k_hbm, v_hbm,
            p_v, k_v, v_v, in_sems, k_sems, v_sems):
    in_cps = [
        pltpu.make_async_copy(
            p_hbm.at[pl.ds(c * _R, _R), :], p_v.at[pl.ds(c * _R, _R), :],
            in_sems.at[c])
        for c in range(_C)
    ]
    for cp in in_cps:
        cp.start()

    z = jnp.zeros((64, 64), jnp.float32)
    wkt = wk_ref[...].T
    wvt = wv_ref[...].T
    wkd = jnp.concatenate(
        [jnp.concatenate([wkt, z], 1), jnp.concatenate([z, wkt], 1)], 0
    )
    wvd = jnp.concatenate(
        [jnp.concatenate([wvt, z], 1), jnp.concatenate([z, wvt], 1)], 0
    )
    bkd = jnp.concatenate([bk_ref[...], bk_ref[...]], 1)
    bvd = jnp.concatenate([bv_ref[...], bv_ref[...]], 1)

    out_cps = []
    for c in range(_C):
        rows = pl.ds(c * _R, _R)
        in_cps[c].wait()
        p = p_v[rows, :]
        k_v[rows, :] = jnp.dot(p, wkd, preferred_element_type=jnp.float32) + bkd
        cpk = pltpu.make_async_copy(
            k_v.at[rows, :], k_hbm.at[rows, :], k_sems.at[c])
        cpk.start()
        v_v[rows, :] = jnp.dot(p, wvd, preferred_element_type=jnp.float32) + bvd
        cpv = pltpu.make_async_copy(
            v_v.at[rows, :], v_hbm.at[rows, :], v_sems.at[c])
        cpv.start()
        out_cps += [cpk, cpv]
    for cp in out_cps:
        cp.wait()


def kernel(x, labels, prototype_vectors, Wk, bk, Wv, bv):
    n, d = prototype_vectors.shape
    a = Wk.shape[0]
    p2 = prototype_vectors.reshape(_N2, 2 * d)
    bk2 = bk.reshape(1, a)
    bv2 = bv.reshape(1, a)
    k2, v2 = pl.pallas_call(
        _encode,
        in_specs=[
            pl.BlockSpec(memory_space=pltpu.MemorySpace.HBM),
            pl.BlockSpec(memory_space=pltpu.MemorySpace.VMEM),
            pl.BlockSpec(memory_space=pltpu.MemorySpace.VMEM),
            pl.BlockSpec(memory_space=pltpu.MemorySpace.VMEM),
            pl.BlockSpec(memory_space=pltpu.MemorySpace.VMEM),
        ],
        out_specs=[
            pl.BlockSpec(memory_space=pltpu.MemorySpace.HBM),
            pl.BlockSpec(memory_space=pltpu.MemorySpace.HBM),
        ],
        out_shape=[
            jax.ShapeDtypeStruct((_N2, 2 * a), jnp.float32),
            jax.ShapeDtypeStruct((_N2, 2 * a), jnp.float32),
        ],
        scratch_shapes=[
            pltpu.VMEM((_N2, 2 * d), jnp.float32),
            pltpu.VMEM((_N2, 2 * a), jnp.float32),
            pltpu.VMEM((_N2, 2 * a), jnp.float32),
            pltpu.SemaphoreType.DMA((_C,)),
            pltpu.SemaphoreType.DMA((_C,)),
            pltpu.SemaphoreType.DMA((_C,)),
        ],
    )(p2, Wk, bk2, Wv, bv2)
    return (k2.reshape(n, a), v2.reshape(n, a))


# speedup vs baseline: 1.0242x; 1.0110x over previous
import jax
import jax.numpy as jnp
from jax.experimental import pallas as pl
from jax.experimental.pallas import tpu as pltpu


def _enc(p_ref, wk_ref, bk_ref, wv_ref, bv_ref, k_ref, v_ref):
    p = p_ref[...]
    z = jnp.zeros((64, 64), jnp.float32)
    wkt = wk_ref[...].T
    wvt = wv_ref[...].T
    wkd = jnp.concatenate([jnp.concatenate([wkt, z], 1), jnp.concatenate([z, wkt], 1)], 0)
    wvd = jnp.concatenate([jnp.concatenate([wvt, z], 1), jnp.concatenate([z, wvt], 1)], 0)
    bkd = jnp.concatenate([bk_ref[...], bk_ref[...]], 1)
    bvd = jnp.concatenate([bv_ref[...], bv_ref[...]], 1)
    k_ref[...] = jnp.dot(p, wkd, preferred_element_type=jnp.float32) + bkd
    v_ref[...] = jnp.dot(p, wvd, preferred_element_type=jnp.float32) + bvd


def kernel(x, labels, prototype_vectors, Wk, bk, Wv, bv):
    p2 = prototype_vectors.reshape(4096, 128)
    vm = pltpu.MemorySpace.VMEM
    k2, v2 = pl.pallas_call(
        _enc,
        in_specs=[pl.BlockSpec(memory_space=vm)] * 5,
        out_specs=[pl.BlockSpec(memory_space=vm), pl.BlockSpec(memory_space=vm)],
        out_shape=[jax.ShapeDtypeStruct((4096, 128), jnp.float32),
                   jax.ShapeDtypeStruct((4096, 128), jnp.float32)],
    )(p2, Wk, bk.reshape(1, 64), Wv, bv.reshape(1, 64))
    return (k2.reshape(8192, 64), v2.reshape(8192, 64))
